# Initial kernel scaffold; baseline (speedup 1.0000x reference)
import jax
import jax.numpy as jnp
from jax.experimental import pallas as pl


def kernel(pairs, heads, relations, tails, ent_emb, rel_emb):
    raise NotImplementedError("write your pallas kernel here")


# speedup vs baseline: 7.8983x; 7.8983x over previous
import functools

import jax
import jax.numpy as jnp
from jax import lax
from jax.experimental import pallas as pl
from jax.experimental.pallas import tpu as pltpu
from jax.experimental.pallas import tpu_sc as plsc

NC = 2
NS = 16
L = 16
NW = NC * NS


def _dg(v, idx):
    return jnp.take_along_axis(v, idx, axis=0, mode="promise_in_bounds")


def _body(ent, relv_h, hbt_h, rels_h, items_h, out_h,
          hbt_v, rels_f, items_v, xrows_v, r00, r01, r10, r11,
          q_f, relv_v, out_v, s00, s01, s10, s11, sem_x,
          *, B, G, M, MP, BW, R):
    wid = lax.axis_index("s") * NC + lax.axis_index("c")
    base = wid * BW
    rows_s = ((r00, r01), (r10, r11))
    sems = ((s00, s01), (s10, s11))
    IOTA = lax.iota(jnp.int32, L)
    SHUF = [IOTA ^ sh for sh in (8, 4, 2, 1)]
    DVEC = [jnp.full((L,), d, jnp.int32) for d in range(L)]

    def bsum(v):
        for sh in SHUF:
            v = v + _dg(v, sh)
        return v

    def bmax(v):
        for sh in SHUF:
            v = jnp.maximum(v, _dg(v, sh))
        return v

    for hop in range(2):
        pltpu.sync_copy(hbt_h.at[pl.ds(hop * B + base, BW)],
                        hbt_v.at[pl.ds(hop * BW, BW)])
        pltpu.sync_copy(rels_h.at[pl.ds((hop * B + base) * MP, BW * MP)],
                        rels_f.at[pl.ds(hop * BW * MP, BW * MP)])
    pltpu.sync_copy(items_h.at[pl.ds(base, BW)], items_v)
    pltpu.sync_copy(relv_h, relv_v)
    pltpu.async_copy(ent.at[items_v], xrows_v, sem_x).wait()

    def fire(b_local, slot):
        for hop in range(2):
            pltpu.async_copy(ent.at[hbt_v.at[hop * BW + b_local]],
                             rows_s[slot][hop], sems[slot][hop])

    def drain(b_local, slot):
        for hop in range(2):
            pltpu.make_async_copy(ent.at[hbt_v.at[hop * BW + b_local]],
                                  rows_s[slot][hop], sems[slot][hop]).wait()

    def compute(b, slot):
        x = xrows_v[b]
        xb = [_dg(x, DVEC[d]) for d in range(L)]
        for k in range(R):
            acc = xb[0] * relv_v[k * L + 0]
            for i in range(1, L):
                acc = acc + xb[i] * relv_v[k * L + i]
            q_f[pl.ds(k * L, L)] = acc
        z = None
        for hop in range(2):
            rows = rows_s[slot][hop]
            row_off = (hop * BW + b) * MP
            hrv = []
            for g in range(G):
                rvec = rels_f[pl.ds(row_off + g * L, L)]
                hg = None
                for j in range(L):
                    m = g * L + j
                    if m >= M:
                        break
                    qrow = q_f[pl.ds(rvec[j] * L, L)]
                    d = bsum(rows[m] * qrow)
                    hg = d if j == 0 else jnp.where(IOTA == j, d, hg)
                hrv.append(hg)
            nreal = M - L * (G - 1)
            if nreal < L:
                hrv[G - 1] = jnp.where(IOTA < nreal, hrv[G - 1],
                                       jnp.float32(-1e30))
            mx = hrv[0]
            for g in range(1, G):
                mx = jnp.maximum(mx, hrv[g])
            mx = bmax(mx)
            es = [jnp.exp(h - mx) for h in hrv]
            s = es[0]
            for g in range(1, G):
                s = s + es[g]
            s = bsum(s)
            pis = [e / s for e in es]
            o = None
            for g in range(G):
                for j in range(L):
                    m = g * L + j
                    if m >= M:
                        break
                    pim = _dg(pis[g], DVEC[j])
                    t = pim * rows[MP + m]
                    o = t if o is None else o + t
            zh = bsum(o * x)
            z = zh if hop == 0 else z + zh
        return z

    fire(0, 0)

    def loop_body(g2, zacc):
        b0 = g2 * 2
        drain(b0, 0)
        fire(b0 + 1, 1)
        z0 = compute(b0, 0)
        lane0 = (b0 % L) + jnp.int32(0)
        zacc = jnp.where(IOTA == lane0, z0, zacc)
        drain(b0 + 1, 1)

        @pl.when(g2 < (BW // 2 - 1))
        def _():
            fire(b0 + 2, 0)

        z1 = compute(b0 + 1, 1)
        zacc = jnp.where(IOTA == lane0 + 1, z1, zacc)

        @pl.when((g2 % (L // 2)) == (L // 2 - 1))
        def _():
            sig = 1.0 / (1.0 + jnp.exp(-zacc))
            out_v[pl.ds(b0 + 2 - L, L)] = sig

        return zacc

    lax.fori_loop(0, BW // 2, loop_body, jnp.zeros((L,), jnp.float32))
    pltpu.sync_copy(out_v, out_h.at[pl.ds(base, BW)])


def kernel(pairs, heads, relations, tails, ent_emb, rel_emb):
    H, B, M = heads.shape
    N_ENT, D = ent_emb.shape
    R = rel_emb.shape[0]
    assert D == L and H == 2 and B % (2 * L * NW) == 0
    G = -(-M // L)
    MP = G * L
    BW = B // NW

    pad = ((0, 0), (0, 0), (0, MP - M))
    heads_p = jnp.pad(heads.astype(jnp.int32), pad)
    tails_p = jnp.pad(tails.astype(jnp.int32), pad)
    rels_p = jnp.pad(relations.astype(jnp.int32), pad).reshape(H * B * MP)
    hbt = jnp.concatenate([heads_p, tails_p], axis=2).reshape(H * B, 2 * MP)
    items = pairs[:, 1].astype(jnp.int32)
    relv = rel_emb.astype(jnp.float32).reshape(R * D, D)

    mesh = plsc.VectorSubcoreMesh(core_axis_name="c", subcore_axis_name="s",
                                  num_cores=NC, num_subcores=NS)
    kfn = pl.kernel(
        functools.partial(_body, B=B, G=G, M=M, MP=MP, BW=BW, R=R),
        out_type=jax.ShapeDtypeStruct((B,), jnp.float32),
        mesh=mesh,
        compiler_params=pltpu.CompilerParams(use_tc_tiling_on_sc=False),
        scratch_types=[
            pltpu.VMEM((2 * BW, 2 * MP), jnp.int32),
            pltpu.VMEM((2 * BW * MP,), jnp.int32),
            pltpu.VMEM((BW,), jnp.int32),
            pltpu.VMEM((BW, D), jnp.float32),
            pltpu.VMEM((2 * MP, D), jnp.float32),
            pltpu.VMEM((2 * MP, D), jnp.float32),
            pltpu.VMEM((2 * MP, D), jnp.float32),
            pltpu.VMEM((2 * MP, D), jnp.float32),
            pltpu.VMEM((R * L,), jnp.float32),
            pltpu.VMEM((R * D, D), jnp.float32),
            pltpu.VMEM((BW,), jnp.float32),
            pltpu.SemaphoreType.DMA,
            pltpu.SemaphoreType.DMA,
            pltpu.SemaphoreType.DMA,
            pltpu.SemaphoreType.DMA,
            pltpu.SemaphoreType.DMA,
        ],
    )
    return kfn(ent_emb, relv, hbt, rels_p, items)
